# baseline (speedup 1.0000x reference)
import functools

import jax
import jax.numpy as jnp
from jax import lax
from jax.experimental import pallas as pl
from jax.experimental.pallas import tpu as pltpu
from jax.experimental.pallas import tpu_sc as plsc

N = 10000
NPAD = 10240
E = 320000
NC, NS, L = 2, 16, 16
NW = NC * NS
K = 128
NCHUNK = 160
EPT = NCHUNK * K
EPAD = NS * EPT
DEG_EPT = EPAD // NW
HW = 64
RPT = NPAD // NS
NBUF = 4

_mesh = plsc.VectorSubcoreMesh(core_axis_name="c", subcore_axis_name="s")
_sc_params = pltpu.CompilerParams(needs_layout_passes=False,
                                  use_tc_tiling_on_sc=False)


@functools.partial(
    pl.kernel,
    out_type=jax.ShapeDtypeStruct((NW, NPAD), jnp.float32),
    mesh=_mesh,
    compiler_params=_sc_params,
    scratch_types=[
        pltpu.VMEM((DEG_EPT,), jnp.int32),
        pltpu.VMEM((NPAD,), jnp.float32),
    ],
)
def _deg_kernel(dst_hbm, part_hbm, dsts, acc):
    c = lax.axis_index("c")
    s = lax.axis_index("s")
    wid = s * NC + c

    @pl.loop(0, NPAD // L)
    def _zero(i):
        acc[pl.ds(i * L, L)] = jnp.zeros((L,), jnp.float32)

    pltpu.sync_copy(dst_hbm.at[pl.ds(wid * DEG_EPT, DEG_EPT)], dsts)
    ones = jnp.ones((L,), jnp.float32)

    @pl.loop(0, DEG_EPT // L)
    def _scatter(j):
        idx = dsts[pl.ds(j * L, L)]
        plsc.addupdate_scatter(acc, [idx], ones)

    pltpu.sync_copy(acc, part_hbm.at[wid])


@functools.partial(
    pl.kernel,
    out_type=jax.ShapeDtypeStruct((NC, NPAD, HW), jnp.float32),
    mesh=_mesh,
    compiler_params=_sc_params,
    scratch_types=[
        pltpu.VMEM((NCHUNK, K), jnp.int32),
        pltpu.VMEM((NCHUNK, K), jnp.int32),
        pltpu.VMEM((NBUF, K, HW), jnp.float32),
        pltpu.VMEM_SHARED((NPAD, HW), jnp.float32),
        [pltpu.SemaphoreType.DMA] * NBUF,
        [pltpu.SemaphoreType.DMA] * NBUF,
        pltpu.SemaphoreType.DMA,
    ],
)
def _scat_kernel(g_hbm, src_hbm, dst_hbm, out_hbm, sidx, didx, rows, accum,
                 sg, ss, sem):
    c = lax.axis_index("c")
    s = lax.axis_index("s")

    ld_s = pltpu.async_copy(src_hbm.at[s], sidx, sem)
    ld_d = pltpu.async_copy(dst_hbm.at[s], didx, sem)
    ld_s.wait()
    ld_d.wait()

    @pl.loop(0, K)
    def _zero_rows(r):
        for j in range(HW // L):
            rows[0, r, pl.ds(j * L, L)] = jnp.zeros((L,), jnp.float32)

    for t in range(RPT // K):
        pltpu.async_copy(rows.at[0], accum.at[pl.ds(s * RPT + t * K, K)], sem)
    for t in range(RPT // K):
        pltpu.make_async_copy(rows.at[0],
                              accum.at[pl.ds(s * RPT + t * K, K)],
                              sem).wait()
    plsc.subcore_barrier()

    def gather_start(jj, b):
        pltpu.async_copy(g_hbm.at[c].at[sidx.at[jj]], rows.at[b], sg[b])

    def gather_wait(jj, b):
        pltpu.make_async_copy(g_hbm.at[c].at[sidx.at[jj]], rows.at[b],
                              sg[b]).wait()

    def scatter_start(jj, b):
        pltpu.async_copy(rows.at[b], accum.at[didx.at[jj]], ss[b], add=True)

    def scatter_wait(jj, b):
        pltpu.make_async_copy(rows.at[b], accum.at[didx.at[jj]],
                              ss[b]).wait()

    gather_start(0, 0)
    gather_start(1, 1)

    @pl.loop(0, NCHUNK, step=NBUF)
    def _edges(j):
        for b in range(NBUF):
            jj = j + b
            nb = (b + 2) % NBUF

            @pl.when(jj >= 2)
            def _():
                scatter_wait(jj - 2, nb)

            @pl.when(jj + 2 < NCHUNK)
            def _():
                gather_start(jj + 2, nb)

            gather_wait(jj, b)
            scatter_start(jj, b)

    scatter_wait(NCHUNK - 2, (NCHUNK - 2) % NBUF)
    scatter_wait(NCHUNK - 1, (NCHUNK - 1) % NBUF)
    plsc.subcore_barrier()

    for t in range(RPT // K):
        off = s * RPT + t * K
        pltpu.async_copy(accum.at[pl.ds(off, K)],
                         out_hbm.at[c].at[pl.ds(off, K)], sem)
    for t in range(RPT // K):
        off = s * RPT + t * K
        pltpu.make_async_copy(accum.at[pl.ds(off, K)],
                              out_hbm.at[c].at[pl.ds(off, K)],
                              sem).wait()


BR = 2048
GRID = NPAD // BR

_H2_SPEC = pl.BlockSpec((NC, BR, HW), lambda i: (0, i, 0))


def _dinv_body(p_ref, o_ref):
    ones = jnp.ones((NW, 1), jnp.float32)
    deg = 1.0 + lax.dot_general(p_ref[...], ones, (((0,), (0,)), ((), ())),
                                preferred_element_type=jnp.float32)
    o_ref[...] = lax.rsqrt(deg)


_dinv_kernel = pl.pallas_call(
    _dinv_body,
    out_shape=jax.ShapeDtypeStruct((NPAD, 1), jnp.float32),
)


def _mm_body(x_ref, w_ref, d_ref, o_ref):
    h = jnp.dot(x_ref[...], w_ref[...], preferred_element_type=jnp.float32,
                precision=lax.Precision.HIGHEST)
    g = d_ref[pl.ds(0, N), :] * h
    for cc in range(NC):
        o_ref[cc, pl.ds(0, N), :] = g[:, cc * HW:(cc + 1) * HW]
        o_ref[cc, pl.ds(N, NPAD - N), :] = jnp.zeros((NPAD - N, HW),
                                                     jnp.float32)


_mm_kernel = pl.pallas_call(
    _mm_body,
    out_shape=jax.ShapeDtypeStruct((NC, NPAD, HW), jnp.float32),
)


def _edges_body(e_ref, o_ref):
    o_ref[:, pl.ds(0, E)] = e_ref[...]
    o_ref[:, pl.ds(E, EPAD - E)] = jnp.full((2, EPAD - E), N, jnp.int32)


_edges_kernel = pl.pallas_call(
    _edges_body,
    out_shape=jax.ShapeDtypeStruct((2, EPAD), jnp.int32),
)


def _wb_body(w2_ref, w3_ref, b1_ref, b2_ref, b3_ref, ws_ref, bs_ref):
    ws_ref[0] = w2_ref[...]
    ws_ref[1, :, pl.ds(0, 64)] = w3_ref[...]
    ws_ref[1, :, pl.ds(64, 64)] = jnp.zeros((128, 64), jnp.float32)
    ws_ref[2] = ws_ref[1]
    bs_ref[0, 0] = b1_ref[...]
    bs_ref[1, 0] = b2_ref[...]
    bs_ref[2, 0, pl.ds(0, 64)] = b3_ref[...]
    bs_ref[2, 0, pl.ds(64, 64)] = jnp.zeros((64,), jnp.float32)


_wb_kernel = pl.pallas_call(
    _wb_body,
    out_shape=(jax.ShapeDtypeStruct((3, 128, 128), jnp.float32),
               jax.ShapeDtypeStruct((3, 1, 128), jnp.float32)),
)


def _comb_body(a_ref, g_ref, d_ref, b_ref, w_ref, o_ref):
    dv = d_ref[...]
    a = jnp.concatenate([a_ref[0], a_ref[1]], axis=-1)
    g = jnp.concatenate([g_ref[0], g_ref[1]], axis=-1)
    pre = dv * (a + g) + b_ref[...]
    x2 = jnp.maximum(pre, 0.0)
    h = jnp.dot(x2, w_ref[...], preferred_element_type=jnp.float32,
                precision=lax.Precision.HIGHEST)
    gn = dv * h
    for cc in range(NC):
        o_ref[cc] = gn[:, cc * HW:(cc + 1) * HW]


_comb128 = pl.pallas_call(
    _comb_body,
    grid=(GRID,),
    in_specs=[
        _H2_SPEC,
        _H2_SPEC,
        pl.BlockSpec((BR, 1), lambda i: (i, 0)),
        pl.BlockSpec((1, 128), lambda i: (0, 0)),
        pl.BlockSpec((128, 128), lambda i: (0, 0)),
    ],
    out_specs=_H2_SPEC,
    out_shape=jax.ShapeDtypeStruct((NC, NPAD, HW), jnp.float32),
)


def _final_body(a_ref, g_ref, d_ref, b_ref, o_ref):
    pre = d_ref[...] * (a_ref[0] + g_ref[0]) + b_ref[...]
    m = jnp.max(pre, axis=1, keepdims=True)
    e = jnp.exp(pre - m)
    lse = jnp.log(jnp.sum(e, axis=1, keepdims=True)) + m
    o_ref[...] = pre - lse


_final_kernel = pl.pallas_call(
    _final_body,
    grid=(GRID,),
    in_specs=[
        _H2_SPEC,
        _H2_SPEC,
        pl.BlockSpec((BR, 1), lambda i: (i, 0)),
        pl.BlockSpec((1, 64), lambda i: (0, 0)),
    ],
    out_specs=pl.BlockSpec((BR, 64), lambda i: (i, 0)),
    out_shape=jax.ShapeDtypeStruct((NPAD, 64), jnp.float32),
)


def kernel(x, edge_index, W1, b1, W2, b2, W3, b3):
    edges = _edges_kernel(edge_index)
    src3 = edges[0].reshape(NS, NCHUNK, K)
    dst3 = edges[1].reshape(NS, NCHUNK, K)

    part = _deg_kernel(edges[1])
    dinv = _dinv_kernel(part)

    g1 = _mm_kernel(x, W1, dinv)

    Ws, bs = _wb_kernel(W2, W3, b1, b2, b3)

    def body(i, carry):
        g, _, _ = carry
        a = _scat_kernel(g, src3, dst3)
        W = lax.dynamic_index_in_dim(Ws, i, 0, keepdims=False)
        b = lax.dynamic_index_in_dim(bs, i, 0, keepdims=False)
        gn = _comb128(a, g, dinv, b, W)
        return (gn, a, g)

    ub = lax.optimization_barrier(jnp.int32(3))
    init = (g1, jnp.zeros((NC, NPAD, HW), jnp.float32), g1)
    _, a, g3 = lax.fori_loop(0, ub, body, init)

    out = _final_kernel(a, g3, dinv, b3.reshape(1, 64))
    return out[:N]


# speedup vs baseline: 1.0145x; 1.0145x over previous
import functools

import jax
import jax.numpy as jnp
from jax import lax
from jax.experimental import pallas as pl
from jax.experimental.pallas import tpu as pltpu
from jax.experimental.pallas import tpu_sc as plsc

N = 10000
NPAD = 10240
E = 320000
NC, NS, L = 2, 16, 16
NW = NC * NS
K = 128
NCHUNK = 160
EPT = NCHUNK * K
EPAD = NS * EPT
DEG_EPT = EPAD // NW
NG = 4
GW = 128 // NG
RPT = NPAD // NS
NBUF = 8
PD = NBUF // 2

_mesh = plsc.VectorSubcoreMesh(core_axis_name="c", subcore_axis_name="s")
_sc_params = pltpu.CompilerParams(needs_layout_passes=False,
                                  use_tc_tiling_on_sc=False)


@functools.partial(
    pl.kernel,
    out_type=jax.ShapeDtypeStruct((NW, NPAD), jnp.float32),
    mesh=_mesh,
    compiler_params=_sc_params,
    scratch_types=[
        pltpu.VMEM((DEG_EPT,), jnp.int32),
        pltpu.VMEM((NPAD,), jnp.float32),
    ],
)
def _deg_kernel(dst_hbm, part_hbm, dsts, acc):
    c = lax.axis_index("c")
    s = lax.axis_index("s")
    wid = s * NC + c

    @pl.loop(0, NPAD // L)
    def _zero(i):
        acc[pl.ds(i * L, L)] = jnp.zeros((L,), jnp.float32)

    pltpu.sync_copy(dst_hbm.at[pl.ds(wid * DEG_EPT, DEG_EPT)], dsts)
    ones = jnp.ones((L,), jnp.float32)

    @pl.loop(0, DEG_EPT // L)
    def _scatter(j):
        idx = dsts[pl.ds(j * L, L)]
        plsc.addupdate_scatter(acc, [idx], ones)

    pltpu.sync_copy(acc, part_hbm.at[wid])


@functools.partial(
    pl.kernel,
    out_type=jax.ShapeDtypeStruct((NG, NPAD, GW), jnp.float32),
    mesh=_mesh,
    compiler_params=_sc_params,
    scratch_types=[
        pltpu.VMEM((NCHUNK, K), jnp.int32),
        pltpu.VMEM((NCHUNK, K), jnp.int32),
        pltpu.VMEM((NBUF, K, GW), jnp.float32),
        pltpu.VMEM_SHARED((NPAD, GW), jnp.float32),
        [pltpu.SemaphoreType.DMA] * NBUF,
        [pltpu.SemaphoreType.DMA] * NBUF,
        pltpu.SemaphoreType.DMA,
    ],
)
def _scat_kernel(g_hbm, src_hbm, dst_hbm, out_hbm, sidx, didx, rows, accum,
                 sg, ss, sem):
    c = lax.axis_index("c")
    s = lax.axis_index("s")

    ld_s = pltpu.async_copy(src_hbm.at[s], sidx, sem)
    ld_d = pltpu.async_copy(dst_hbm.at[s], didx, sem)
    ld_s.wait()
    ld_d.wait()

    def gather_start(q, jj, b):
        pltpu.async_copy(g_hbm.at[q].at[sidx.at[jj]], rows.at[b], sg[b])

    def gather_wait(q, jj, b):
        pltpu.make_async_copy(g_hbm.at[q].at[sidx.at[jj]], rows.at[b],
                              sg[b]).wait()

    def scatter_start(jj, b):
        pltpu.async_copy(rows.at[b], accum.at[didx.at[jj]], ss[b], add=True)

    def scatter_wait(jj, b):
        pltpu.make_async_copy(rows.at[b], accum.at[didx.at[jj]], ss[b]).wait()

    for p in range(2):
        q = c * 2 + p

        @pl.loop(0, K)
        def _zero_rows(r):
            for j in range(GW // L):
                rows[0, r, pl.ds(j * L, L)] = jnp.zeros((L,), jnp.float32)

        for t in range(RPT // K):
            pltpu.async_copy(rows.at[0], accum.at[pl.ds(s * RPT + t * K, K)],
                             sem)
        for t in range(RPT // K):
            pltpu.make_async_copy(rows.at[0],
                                  accum.at[pl.ds(s * RPT + t * K, K)],
                                  sem).wait()
        plsc.subcore_barrier()

        for b0 in range(PD):
            gather_start(q, b0, b0)

        @pl.loop(0, NCHUNK, step=NBUF)
        def _edges(j):
            for b in range(NBUF):
                jj = j + b
                nb = (b + PD) % NBUF

                @pl.when(jj >= PD)
                def _():
                    scatter_wait(jj - PD, nb)

                @pl.when(jj + PD < NCHUNK)
                def _():
                    gather_start(q, jj + PD, nb)

                gather_wait(q, jj, b)
                scatter_start(jj, b)

        for t in range(NCHUNK - PD, NCHUNK):
            scatter_wait(t, t % NBUF)
        plsc.subcore_barrier()

        for t in range(RPT // K):
            off = s * RPT + t * K
            pltpu.async_copy(accum.at[pl.ds(off, K)],
                             out_hbm.at[q].at[pl.ds(off, K)], sem)
        for t in range(RPT // K):
            off = s * RPT + t * K
            pltpu.make_async_copy(accum.at[pl.ds(off, K)],
                                  out_hbm.at[q].at[pl.ds(off, K)],
                                  sem).wait()


BR = 2048
GRID = NPAD // BR

_G4_SPEC = pl.BlockSpec((NG, BR, GW), lambda i: (0, i, 0))


def _dinv_body(p_ref, o_ref):
    ones = jnp.ones((NW, 1), jnp.float32)
    deg = 1.0 + lax.dot_general(p_ref[...], ones, (((0,), (0,)), ((), ())),
                                preferred_element_type=jnp.float32)
    o_ref[...] = lax.rsqrt(deg)


_dinv_kernel = pl.pallas_call(
    _dinv_body,
    out_shape=jax.ShapeDtypeStruct((NPAD, 1), jnp.float32),
)


def _mm_body(x_ref, w_ref, d_ref, o_ref):
    h = jnp.dot(x_ref[...], w_ref[...], preferred_element_type=jnp.float32,
                precision=lax.Precision.HIGHEST)
    g = d_ref[pl.ds(0, N), :] * h
    for qq in range(NG):
        o_ref[qq, pl.ds(0, N), :] = g[:, qq * GW:(qq + 1) * GW]
        o_ref[qq, pl.ds(N, NPAD - N), :] = jnp.zeros((NPAD - N, GW),
                                                     jnp.float32)


_mm_kernel = pl.pallas_call(
    _mm_body,
    out_shape=jax.ShapeDtypeStruct((NG, NPAD, GW), jnp.float32),
)


def _edges_body(e_ref, o_ref):
    o_ref[:, pl.ds(0, E)] = e_ref[...]
    o_ref[:, pl.ds(E, EPAD - E)] = jnp.full((2, EPAD - E), N, jnp.int32)


_edges_kernel = pl.pallas_call(
    _edges_body,
    out_shape=jax.ShapeDtypeStruct((2, EPAD), jnp.int32),
)


def _wb_body(w2_ref, w3_ref, b1_ref, b2_ref, b3_ref, ws_ref, bs_ref):
    ws_ref[0] = w2_ref[...]
    ws_ref[1, :, pl.ds(0, 64)] = w3_ref[...]
    ws_ref[1, :, pl.ds(64, 64)] = jnp.zeros((128, 64), jnp.float32)
    ws_ref[2] = ws_ref[1]
    bs_ref[0, 0] = b1_ref[...]
    bs_ref[1, 0] = b2_ref[...]
    bs_ref[2, 0, pl.ds(0, 64)] = b3_ref[...]
    bs_ref[2, 0, pl.ds(64, 64)] = jnp.zeros((64,), jnp.float32)


_wb_kernel = pl.pallas_call(
    _wb_body,
    out_shape=(jax.ShapeDtypeStruct((3, 128, 128), jnp.float32),
               jax.ShapeDtypeStruct((3, 1, 128), jnp.float32)),
)


def _join(ref4):
    return jnp.concatenate([ref4[qq] for qq in range(NG)], axis=-1)


def _comb_body(a_ref, g_ref, d_ref, b_ref, w_ref, o_ref):
    dv = d_ref[...]
    pre = dv * (_join(a_ref) + _join(g_ref)) + b_ref[...]
    x2 = jnp.maximum(pre, 0.0)
    h = jnp.dot(x2, w_ref[...], preferred_element_type=jnp.float32,
                precision=lax.Precision.HIGHEST)
    gn = dv * h
    for qq in range(NG):
        o_ref[qq] = gn[:, qq * GW:(qq + 1) * GW]


_comb128 = pl.pallas_call(
    _comb_body,
    grid=(GRID,),
    in_specs=[
        _G4_SPEC,
        _G4_SPEC,
        pl.BlockSpec((BR, 1), lambda i: (i, 0)),
        pl.BlockSpec((1, 128), lambda i: (0, 0)),
        pl.BlockSpec((128, 128), lambda i: (0, 0)),
    ],
    out_specs=_G4_SPEC,
    out_shape=jax.ShapeDtypeStruct((NG, NPAD, GW), jnp.float32),
)


def _final_body(a_ref, g_ref, d_ref, b_ref, o_ref):
    a01 = jnp.concatenate([a_ref[0], a_ref[1]], axis=-1)
    g01 = jnp.concatenate([g_ref[0], g_ref[1]], axis=-1)
    pre = d_ref[...] * (a01 + g01) + b_ref[...]
    m = jnp.max(pre, axis=1, keepdims=True)
    e = jnp.exp(pre - m)
    lse = jnp.log(jnp.sum(e, axis=1, keepdims=True)) + m
    o_ref[...] = pre - lse


_final_kernel = pl.pallas_call(
    _final_body,
    grid=(GRID,),
    in_specs=[
        _G4_SPEC,
        _G4_SPEC,
        pl.BlockSpec((BR, 1), lambda i: (i, 0)),
        pl.BlockSpec((1, 64), lambda i: (0, 0)),
    ],
    out_specs=pl.BlockSpec((BR, 64), lambda i: (i, 0)),
    out_shape=jax.ShapeDtypeStruct((NPAD, 64), jnp.float32),
)


def kernel(x, edge_index, W1, b1, W2, b2, W3, b3):
    edges = _edges_kernel(edge_index)
    src3 = edges[0].reshape(NS, NCHUNK, K)
    dst3 = edges[1].reshape(NS, NCHUNK, K)

    part = _deg_kernel(edges[1])
    dinv = _dinv_kernel(part)

    g1 = _mm_kernel(x, W1, dinv)

    Ws, bs = _wb_kernel(W2, W3, b1, b2, b3)

    def body(i, carry):
        g, _, _ = carry
        a = _scat_kernel(g, src3, dst3)
        W = lax.dynamic_index_in_dim(Ws, i, 0, keepdims=False)
        b = lax.dynamic_index_in_dim(bs, i, 0, keepdims=False)
        gn = _comb128(a, g, dinv, b, W)
        return (gn, a, g)

    ub = lax.optimization_barrier(jnp.int32(3))
    init = (g1, jnp.zeros((NG, NPAD, GW), jnp.float32), g1)
    _, a, g3 = lax.fori_loop(0, ub, body, init)

    out = _final_kernel(a, g3, dinv, b3.reshape(1, 64))
    return out[:N]
